# baseline (speedup 1.0000x reference)
import jax
import jax.numpy as jnp
from jax import lax
from jax.experimental import pallas as pl
from jax.experimental.pallas import tpu as pltpu
from jax.experimental.pallas import tpu_sc as plsc

NCLS = 1000
NCPAD = 1024
NY = 320000
D = 128
NC, NS = 2, 16
NW = NC * NS
CHUNK = 256
SUB = 128
NSUB = CHUNK // SUB
TOTAL_CHUNKS = NY // CHUNK
BASE_CH, EXTRA = divmod(TOTAL_CHUNKS, NW)
MAX_CH = BASE_CH + 1
HCHUNK = 12800
HSTEPS = NY // HCHUNK
HW = 128
XB = 1024


def _sc_seg_body(y_h, lab_h, zacc_h,
                 sums_h,
                 rows_v, idx_v, acc_sh, isem0, isem1, osem0, osem1):
    c = lax.axis_index("c")
    s = lax.axis_index("s")
    w = s * NC + c
    isem = [isem0, isem1]
    osem = [osem0, osem1]

    @pl.when(s == 0)
    def _init():
        pltpu.sync_copy(zacc_h, acc_sh)

    plsc.subcore_barrier()

    n_ch = BASE_CH + (w < EXTRA).astype(jnp.int32)
    first = w * BASE_CH + jnp.minimum(w, EXTRA)

    def start_in(k, b):
        g = first + k
        pltpu.async_copy(y_h.at[pl.ds(g * CHUNK, CHUNK)], rows_v.at[b],
                         isem[b])
        pltpu.async_copy(lab_h.at[pl.ds(g * NSUB, NSUB)], idx_v.at[b],
                         isem[b])

    def wait_in(k, b):
        g = first + k
        pltpu.make_async_copy(y_h.at[pl.ds(g * CHUNK, CHUNK)], rows_v.at[b],
                              isem[b]).wait()
        pltpu.make_async_copy(lab_h.at[pl.ds(g * NSUB, NSUB)], idx_v.at[b],
                              isem[b]).wait()

    def start_scat(b):
        for j in range(NSUB):
            pltpu.async_copy(rows_v.at[b, pl.ds(j * SUB, SUB)],
                             acc_sh.at[idx_v.at[b, j]], osem[b], add=True)

    def wait_scat(b):
        for j in range(NSUB):
            pltpu.make_async_copy(rows_v.at[b, pl.ds(j * SUB, SUB)],
                                  acc_sh.at[idx_v.at[b, j]],
                                  osem[b]).wait()

    @pl.when(0 < n_ch)
    def _prime():
        start_in(0, 0)

    def pair(p, carry):
        for b in range(2):
            k = 2 * p + b

            @pl.when((k >= 1) & (k - 1 < n_ch))
            def _w():
                wait_scat(1 - b)

            @pl.when(k + 1 < n_ch)
            def _s():
                start_in(k + 1, 1 - b)

            @pl.when(k < n_ch)
            def _go():
                wait_in(k, b)
                start_scat(b)
        return carry

    lax.fori_loop(0, MAX_CH // 2, pair, 0)

    @pl.when(n_ch == MAX_CH)
    def _tail():
        wait_scat((MAX_CH - 1) % 2)

    plsc.subcore_barrier()

    @pl.when(s == 0)
    def _writeout():
        pltpu.sync_copy(acc_sh, sums_h.at[c])


def _tc_body(lb, xb, sums, out, cnts):
    i = pl.program_id(0)

    @pl.when(i == 0)
    def _init():
        cnts[...] = jnp.zeros_like(cnts)

    @pl.when(i < HSTEPS)
    def _hist():
        lab = lb[pl.ds(jnp.minimum(i, HSTEPS - 1), 1), :]
        lab0 = lab[0, 0]
        lab_last = lab[0, HCHUNK - 1]
        base = jnp.minimum(lab0 & ~7, NCPAD - HW)
        narrow = (lab_last - base) < HW

        @pl.when(narrow)
        def _windowed():
            oh_t = (jnp.broadcast_to(lab, (HW, HCHUNK))
                    == jax.lax.broadcasted_iota(jnp.int32, (HW, HCHUNK), 0)
                    + base).astype(jnp.float32)
            cnts[pl.ds(base, HW), :] += jnp.broadcast_to(
                jnp.sum(oh_t, axis=1, keepdims=True), (HW, 8))

        @pl.when(jnp.logical_not(narrow))
        def _full():
            oh_t = (jnp.broadcast_to(lab, (NCPAD, HCHUNK))
                    == jax.lax.broadcasted_iota(jnp.int32, (NCPAD, HCHUNK), 0)
                    ).astype(jnp.float32)
            cnts[...] += jnp.broadcast_to(
                jnp.sum(oh_t, axis=1, keepdims=True), (NCPAD, 8))

    @pl.when(i >= HSTEPS)
    def _softmax():
        ssum = sums[0] + sums[1]
        cnt = jnp.maximum(cnts[0:NCLS, 0:1], 1.0)
        cluster = ssum / cnt
        logits = jax.lax.dot_general(xb[...], cluster,
                                     (((1,), (1,)), ((), ())),
                                     preferred_element_type=jnp.float32)
        m = jnp.max(logits, axis=1, keepdims=True)
        e = jnp.exp(logits - m)
        out[...] = e / jnp.sum(e, axis=1, keepdims=True)


def kernel(x, y, y_label):
    labels = y_label.astype(jnp.int32)
    lab2 = labels.reshape(NY // SUB, SUB)
    lab2b = labels.reshape(HSTEPS, HCHUNK)
    zacc = jnp.zeros((NCLS, D), jnp.float32)

    seg = pl.kernel(
        _sc_seg_body,
        out_type=jax.ShapeDtypeStruct((NC, NCLS, D), jnp.float32),
        mesh=plsc.VectorSubcoreMesh(core_axis_name="c", subcore_axis_name="s"),
        scratch_types=[
            pltpu.VMEM((2, CHUNK, D), jnp.float32),
            pltpu.VMEM((2, NSUB, SUB), jnp.int32),
            pltpu.VMEM_SHARED((NCLS, D), jnp.float32),
            pltpu.SemaphoreType.DMA,
            pltpu.SemaphoreType.DMA,
            pltpu.SemaphoreType.DMA,
            pltpu.SemaphoreType.DMA,
        ],
    )
    sums = seg(y, lab2, zacc)

    nxb = x.shape[0] // XB
    probs = pl.pallas_call(
        _tc_body,
        grid=(HSTEPS + nxb,),
        in_specs=[
            pl.BlockSpec((HSTEPS, HCHUNK), lambda i: (0, 0)),
            pl.BlockSpec((XB, D), lambda i: (jnp.maximum(i - HSTEPS, 0), 0)),
            pl.BlockSpec((NC, NCLS, D), lambda i: (0, 0, 0)),
        ],
        out_specs=pl.BlockSpec((XB, NCLS),
                               lambda i: (jnp.maximum(i - HSTEPS, 0), 0)),
        out_shape=jax.ShapeDtypeStruct((x.shape[0], NCLS), jnp.float32),
        scratch_shapes=[pltpu.VMEM((NCPAD, 8), jnp.float32)],
    )(lab2b, x, sums)

    return probs


# speedup vs baseline: 1.1649x; 1.1649x over previous
import jax
import jax.numpy as jnp
from jax import lax
from jax.experimental import pallas as pl
from jax.experimental.pallas import tpu as pltpu
from jax.experimental.pallas import tpu_sc as plsc

NCLS = 1000
NCPAD = 1024
NY = 320000
D = 128
NC, NS = 2, 16
NW = NC * NS
CHUNK = 256
SUB = 128
NSUB = CHUNK // SUB
TOTAL_CHUNKS = NY // CHUNK
BASE_CH, EXTRA = divmod(TOTAL_CHUNKS, NW)
MAX_CH = BASE_CH + 1
HCHUNK = 12800
HSTEPS = NY // HCHUNK
HW = 128
XB = 1024


def _sc_seg_body(y_h, lab_h, zacc_h,
                 sums_h,
                 rows_v, idx_v, acc_sh, isem0, isem1, osem0, osem1):
    c = lax.axis_index("c")
    s = lax.axis_index("s")
    w = s * NC + c
    isem = [isem0, isem1]
    osem = [osem0, osem1]

    @pl.when(s == 0)
    def _init():
        pltpu.sync_copy(zacc_h, acc_sh)

    plsc.subcore_barrier()

    n_ch = BASE_CH + (w < EXTRA).astype(jnp.int32)
    first = w * BASE_CH + jnp.minimum(w, EXTRA)

    def start_in(k, b):
        g = first + k
        pltpu.async_copy(y_h.at[pl.ds(g * CHUNK, CHUNK)], rows_v.at[b],
                         isem[b])
        pltpu.async_copy(lab_h.at[pl.ds(g * NSUB, NSUB)], idx_v.at[b],
                         isem[b])

    def wait_in(k, b):
        g = first + k
        pltpu.make_async_copy(y_h.at[pl.ds(g * CHUNK, CHUNK)], rows_v.at[b],
                              isem[b]).wait()
        pltpu.make_async_copy(lab_h.at[pl.ds(g * NSUB, NSUB)], idx_v.at[b],
                              isem[b]).wait()

    def start_scat(b):
        for j in range(NSUB):
            pltpu.async_copy(rows_v.at[b, pl.ds(j * SUB, SUB)],
                             acc_sh.at[idx_v.at[b, j]], osem[b], add=True)

    def wait_scat(b):
        for j in range(NSUB):
            pltpu.make_async_copy(rows_v.at[b, pl.ds(j * SUB, SUB)],
                                  acc_sh.at[idx_v.at[b, j]],
                                  osem[b]).wait()

    @pl.when(0 < n_ch)
    def _prime():
        start_in(0, 0)

    def pair(p, carry):
        for b in range(2):
            k = 2 * p + b

            @pl.when((k >= 1) & (k - 1 < n_ch))
            def _w():
                wait_scat(1 - b)

            @pl.when(k + 1 < n_ch)
            def _s():
                start_in(k + 1, 1 - b)

            @pl.when(k < n_ch)
            def _go():
                wait_in(k, b)
                start_scat(b)
        return carry

    lax.fori_loop(0, MAX_CH // 2, pair, 0)

    @pl.when(n_ch == MAX_CH)
    def _tail():
        wait_scat((MAX_CH - 1) % 2)

    plsc.subcore_barrier()

    @pl.when(s == 0)
    def _writeout():
        pltpu.sync_copy(acc_sh, sums_h.at[c])


def _hist_body(lb, cnts):
    i = pl.program_id(0)

    @pl.when(i == 0)
    def _init():
        cnts[...] = jnp.zeros_like(cnts)

    lab = lb[pl.ds(i, 1), :]
    lab0 = lab[0, 0]
    lab_last = lab[0, HCHUNK - 1]
    base = jnp.minimum(lab0 & ~7, NCPAD - HW)
    narrow = (lab_last - base) < HW

    @pl.when(narrow)
    def _windowed():
        oh_t = (jnp.broadcast_to(lab, (HW, HCHUNK))
                == jax.lax.broadcasted_iota(jnp.int32, (HW, HCHUNK), 0) + base
                ).astype(jnp.float32)
        cnts[pl.ds(base, HW), :] += jnp.broadcast_to(
            jnp.sum(oh_t, axis=1, keepdims=True), (HW, 8))

    @pl.when(jnp.logical_not(narrow))
    def _full():
        oh_t = (jnp.broadcast_to(lab, (NCPAD, HCHUNK))
                == jax.lax.broadcasted_iota(jnp.int32, (NCPAD, HCHUNK), 0)
                ).astype(jnp.float32)
        cnts[...] += jnp.broadcast_to(jnp.sum(oh_t, axis=1, keepdims=True),
                                      (NCPAD, 8))


def _sm_body(xb, sums, cnts, out):
    ssum = sums[0] + sums[1]
    cnt = jnp.maximum(cnts[0:NCLS, 0:1], 1.0)
    cluster = ssum / cnt
    logits = jax.lax.dot_general(xb[...], cluster, (((1,), (1,)), ((), ())),
                                 preferred_element_type=jnp.float32)
    m = jnp.max(logits, axis=1, keepdims=True)
    e = jnp.exp(logits - m)
    out[...] = e / jnp.sum(e, axis=1, keepdims=True)


def kernel(x, y, y_label):
    labels = y_label.astype(jnp.int32)
    lab2 = labels.reshape(NY // SUB, SUB)
    lab2b = labels.reshape(HSTEPS, HCHUNK)
    zacc = jnp.zeros((NCLS, D), jnp.float32)

    seg = pl.kernel(
        _sc_seg_body,
        out_type=jax.ShapeDtypeStruct((NC, NCLS, D), jnp.float32),
        mesh=plsc.VectorSubcoreMesh(core_axis_name="c", subcore_axis_name="s"),
        scratch_types=[
            pltpu.VMEM((2, CHUNK, D), jnp.float32),
            pltpu.VMEM((2, NSUB, SUB), jnp.int32),
            pltpu.VMEM_SHARED((NCLS, D), jnp.float32),
            pltpu.SemaphoreType.DMA,
            pltpu.SemaphoreType.DMA,
            pltpu.SemaphoreType.DMA,
            pltpu.SemaphoreType.DMA,
        ],
    )
    sums = seg(y, lab2, zacc)

    cnts = pl.pallas_call(
        _hist_body,
        grid=(HSTEPS,),
        in_specs=[pl.BlockSpec((HSTEPS, HCHUNK), lambda i: (0, 0))],
        out_specs=pl.BlockSpec((NCPAD, 8), lambda i: (0, 0)),
        out_shape=jax.ShapeDtypeStruct((NCPAD, 8), jnp.float32),
    )(lab2b)

    probs = pl.pallas_call(
        _sm_body,
        grid=(x.shape[0] // XB,),
        in_specs=[
            pl.BlockSpec((XB, D), lambda i: (i, 0)),
            pl.BlockSpec((NC, NCLS, D), lambda i: (0, 0, 0)),
            pl.BlockSpec((NCPAD, 8), lambda i: (0, 0)),
        ],
        out_specs=pl.BlockSpec((XB, NCLS), lambda i: (i, 0)),
        out_shape=jax.ShapeDtypeStruct((x.shape[0], NCLS), jnp.float32),
    )(x, sums, cnts)

    return probs
